# baseline (speedup 1.0000x reference)
import functools

import jax
import jax.numpy as jnp
from jax import lax
from jax.experimental import pallas as pl
from jax.experimental.pallas import tpu as pltpu
from jax.experimental.pallas import tpu_sc as plsc

N = 100000
G = 1000
E = 3200000
RB = 2000
NBLK = N // RB

_f32 = jnp.float32


def _row_specs(*dims):
    return [pl.BlockSpec((RB, d), lambda i: (i, 0)) for d in dims]


def _split_spec(d=16):
    return pl.BlockSpec((2, RB, d), lambda i: (0, i, 0))


def _full_spec(shape):
    nd = len(shape)
    return pl.BlockSpec(shape, lambda i: (0,) * nd)


def _dot(a, b):
    return jax.lax.dot_general(a, b, (((1,), (0,)), ((), ())),
                               preferred_element_type=_f32,
                               precision=jax.lax.Precision.DEFAULT)


def _prep_body(dp_ref, x_ref, dinv_ref, xs0_ref):
    deg = dp_ref[0] + dp_ref[1] + 1.0
    dinv = lax.rsqrt(deg)
    dinv_ref[...] = dinv
    xs = x_ref[...] * dinv
    xs0_ref[...] = jnp.concatenate(
        [xs, jnp.zeros((RB, 11), _f32)], axis=1)


def _prep(deg_parts, x):
    return pl.pallas_call(
        _prep_body,
        grid=(NBLK,),
        in_specs=[_split_spec(1)] + _row_specs(5),
        out_specs=_row_specs(1, 16),
        out_shape=[jax.ShapeDtypeStruct((N, 1), _f32),
                   jax.ShapeDtypeStruct((N, 16), _f32)],
    )(deg_parts, x)


def _layer16_body(acc_ref, xs_ref, dinv_ref, W_ref, b_ref, out_ref):
    dinv = dinv_ref[...]
    t = dinv * (acc_ref[0] + acc_ref[1] + xs_ref[...])
    h = jnp.maximum(_dot(t, W_ref[...]) + b_ref[...], 0.0)
    xso = dinv * h
    out_ref[0] = xso[:, :16]
    out_ref[1] = xso[:, 16:]


def _layer16(acc_parts, xs, dinv, Wp, b):
    return pl.pallas_call(
        _layer16_body,
        grid=(NBLK,),
        in_specs=[_split_spec(), *_row_specs(16, 1),
                  _full_spec((16, 32)), _full_spec((1, 32))],
        out_specs=_split_spec(),
        out_shape=jax.ShapeDtypeStruct((2, N, 16), _f32),
    )(acc_parts, xs, dinv, Wp, b)


def _layer32_body(acc_ref, xs_ref, dinv_ref, W_ref, b_ref, out_ref):
    dinv = dinv_ref[...]
    s = jnp.concatenate([acc_ref[0], acc_ref[1]], axis=1)
    xsc = jnp.concatenate([xs_ref[0], xs_ref[1]], axis=1)
    t = dinv * (s + xsc)
    h = jnp.maximum(_dot(t, W_ref[...]) + b_ref[...], 0.0)
    xso = dinv * h
    out_ref[0] = xso[:, :16]
    out_ref[1] = xso[:, 16:]


def _layer32(acc, xs, dinv, W, b):
    return pl.pallas_call(
        _layer32_body,
        grid=(NBLK,),
        in_specs=[_split_spec(), _split_spec(), *_row_specs(1),
                  _full_spec((32, 32)), _full_spec((1, 32))],
        out_specs=_split_spec(),
        out_shape=jax.ShapeDtypeStruct((2, N, 16), _f32),
    )(acc, xs, dinv, W, b)


def _enc4_body(acc_ref, xs_ref, dinv_ref, W_ref, b_ref,
               mW1_ref, mb1_ref, mW2_ref, mb2_ref,
               gW1_ref, gb1_ref, gW2_ref, gb2_ref, out_ref):
    dinv = dinv_ref[...]
    s = jnp.concatenate([acc_ref[0], acc_ref[1]], axis=1)
    xsc = jnp.concatenate([xs_ref[0], xs_ref[1]], axis=1)
    t = dinv * (s + xsc)
    h = jnp.maximum(_dot(t, W_ref[...]) + b_ref[...], 0.0)
    m = _dot(jnp.maximum(_dot(h, mW1_ref[...]) + mb1_ref[...], 0.0),
             mW2_ref[...]) + mb2_ref[...]
    g = _dot(jnp.maximum(_dot(h, gW1_ref[...]) + gb1_ref[...], 0.0),
             gW2_ref[...]) + gb2_ref[...]
    out_ref[...] = jnp.concatenate(
        [m, g, jnp.ones((RB, 1), _f32), jnp.zeros((RB, 9), _f32)], axis=1)


def _enc4_head(acc, xs, dinv, W, b, mW1, mb1, mW2, mb2, gW1, gb1, gW2, gb2):
    return pl.pallas_call(
        _enc4_body,
        grid=(NBLK,),
        in_specs=[_split_spec(), _split_spec(), *_row_specs(1),
                  _full_spec((32, 32)), _full_spec((1, 32)),
                  _full_spec((32, 16)), _full_spec((1, 16)),
                  _full_spec((16, 3)), _full_spec((1, 3)),
                  _full_spec((32, 16)), _full_spec((1, 16)),
                  _full_spec((16, 3)), _full_spec((1, 3))],
        out_specs=_row_specs(16)[0],
        out_shape=jax.ShapeDtypeStruct((N, 16), _f32),
    )(acc, xs, dinv, W, b, mW1, mb1, mW2, mb2, gW1, gb1, gW2, gb2)


def _z_body(pool_ref, eps_ref, mu_ref, sg_ref, zp_ref):
    p = pool_ref[0] + pool_ref[1]
    denom = jnp.maximum(p[:, 6:7], 1.0)
    mu = p[:, 0:3] / denom
    sg = p[:, 3:6] / denom
    z = mu + eps_ref[...] * jnp.exp(0.5 * sg)
    mu_ref[...] = mu
    sg_ref[...] = sg
    zp_ref[...] = jnp.concatenate([z, jnp.zeros((G, 13), _f32)], axis=1)


def _z_kernel(pooled, eps):
    return pl.pallas_call(
        _z_body,
        in_specs=[pl.BlockSpec((2, G, 16), lambda: (0, 0, 0)),
                  pl.BlockSpec((G, 3), lambda: (0, 0))],
        out_specs=[pl.BlockSpec((G, 3), lambda: (0, 0)),
                   pl.BlockSpec((G, 3), lambda: (0, 0)),
                   pl.BlockSpec((G, 16), lambda: (0, 0))],
        out_shape=[jax.ShapeDtypeStruct((G, 3), _f32),
                   jax.ShapeDtypeStruct((G, 3), _f32),
                   jax.ShapeDtypeStruct((G, 16), _f32)],
    )(pooled, eps)


def _dec_head_body(zn_ref, dinv_ref, W1_ref, b1_ref, W2_ref, b2_ref, out_ref):
    h = jnp.maximum(_dot(zn_ref[...], W1_ref[...]) + b1_ref[...], 0.0)
    h2 = jnp.maximum(_dot(h, W2_ref[...]) + b2_ref[...], 0.0)
    xso = dinv_ref[...] * h2
    out_ref[0] = xso[:, :16]
    out_ref[1] = xso[:, 16:]


def _dec_head(zn, dinv, W1p, b1, W2, b2):
    return pl.pallas_call(
        _dec_head_body,
        grid=(NBLK,),
        in_specs=[*_row_specs(16, 1), _full_spec((16, 16)),
                  _full_spec((1, 16)), _full_spec((16, 32)),
                  _full_spec((1, 32))],
        out_specs=_split_spec(),
        out_shape=jax.ShapeDtypeStruct((2, N, 16), _f32),
    )(zn, dinv, W1p, b1, W2, b2)


def _dec3_body(acc_ref, xs_ref, dinv_ref, W_ref, b_ref, W4_ref, out_ref):
    dinv = dinv_ref[...]
    s = jnp.concatenate([acc_ref[0], acc_ref[1]], axis=1)
    xsc = jnp.concatenate([xs_ref[0], xs_ref[1]], axis=1)
    t = dinv * (s + xsc)
    h = jnp.maximum(_dot(t, W_ref[...]) + b_ref[...], 0.0)
    out_ref[...] = _dot(dinv * h, W4_ref[...])


def _dec3(acc, xs, dinv, W3, b3, W4p):
    return pl.pallas_call(
        _dec3_body,
        grid=(NBLK,),
        in_specs=[_split_spec(), _split_spec(), *_row_specs(1),
                  _full_spec((32, 32)), _full_spec((1, 32)),
                  _full_spec((32, 16))],
        out_specs=_row_specs(16)[0],
        out_shape=jax.ShapeDtypeStruct((N, 16), _f32),
    )(acc, xs, dinv, W3, b3, W4p)


def _final_body(acc_ref, ys_ref, dinv_ref, b_ref, out_ref):
    t = dinv_ref[...] * (acc_ref[0] + acc_ref[1] + ys_ref[...]) + b_ref[...]
    out_ref[...] = jnp.maximum(t, 0.0)[:, :5]


def _final(acc_parts, ys, dinv, b4p):
    return pl.pallas_call(
        _final_body,
        grid=(NBLK,),
        in_specs=[_split_spec(), *_row_specs(16, 1), _full_spec((1, 16))],
        out_specs=_row_specs(5)[0],
        out_shape=jax.ShapeDtypeStruct((N, 5), _f32),
    )(acc_parts, ys, dinv, b4p)


def _deg_parts(src, dst):
    d = jnp.zeros((N,), _f32).at[dst].add(1.0)
    return jnp.stack([d, jnp.zeros((N,), _f32)]).reshape(2, N, 1)


def _agg16(xs, src, dst):
    s = jnp.zeros((N, 16), _f32).at[dst].add(xs[src])
    return jnp.stack([s, jnp.zeros((N, 16), _f32)])


def _agg32(xs_halves, src, dst):
    f = lambda xh: jnp.zeros((N, 16), _f32).at[dst].add(xh[src])
    return jnp.stack([f(xs_halves[0]), f(xs_halves[1])])


def _pool(combined, batch_index):
    p = jnp.zeros((G, 16), _f32).at[batch_index].add(combined)
    return jnp.stack([p, jnp.zeros((G, 16), _f32)])


def _zn_gather(zpad, batch_index):
    return zpad[batch_index]


def kernel(x, edge_index, batch_index,
           enc_W1, enc_b1, enc_W2, enc_b2, enc_W3, enc_b3, enc_W4, enc_b4,
           mu_W1, mu_b1, mu_W2, mu_b2,
           sg_W1, sg_b1, sg_W2, sg_b2,
           un_W1, un_b1, un_W2, un_b2,
           dec_W1, dec_b1, dec_W2, dec_b2, dec_W3, dec_b3, dec_W4, dec_b4):
    src = edge_index[0].astype(jnp.int32)
    dst = edge_index[1].astype(jnp.int32)

    W1p = jnp.zeros((16, 32), _f32).at[:5].set(enc_W1)
    unW1p = jnp.zeros((16, 16), _f32).at[:3].set(un_W1)
    decW4p = jnp.zeros((32, 16), _f32).at[:, :5].set(dec_W4)
    b4p = jnp.zeros((1, 16), _f32).at[0, :5].set(dec_b4)
    eps = jax.random.normal(jax.random.key(42), (G, 3), dtype=_f32)

    deg_parts = _deg_parts(src, dst)
    dinv, xs0 = _prep(deg_parts, x)

    acc = _agg16(xs0, src, dst)
    xs = _layer16(acc, xs0, dinv, W1p, enc_b1.reshape(1, 32))
    for W, b in ((enc_W2, enc_b2), (enc_W3, enc_b3)):
        acc = _agg32(xs, src, dst)
        xs = _layer32(acc, xs, dinv, W, b.reshape(1, 32))
    acc = _agg32(xs, src, dst)
    combined = _enc4_head(acc, xs, dinv, enc_W4, enc_b4.reshape(1, 32),
                          mu_W1, mu_b1.reshape(1, 16), mu_W2,
                          mu_b2.reshape(1, 3),
                          sg_W1, sg_b1.reshape(1, 16), sg_W2,
                          sg_b2.reshape(1, 3))

    pooled = _pool(combined, batch_index)
    mu, sigma, zpad = _z_kernel(pooled, eps)

    zn = _zn_gather(zpad, batch_index)
    xs = _dec_head(zn, dinv, unW1p, un_b1.reshape(1, 16), un_W2,
                   un_b2.reshape(1, 32))
    for W, b in ((dec_W1, dec_b1), (dec_W2, dec_b2)):
        acc = _agg32(xs, src, dst)
        xs = _layer32(acc, xs, dinv, W, b.reshape(1, 32))
    acc = _agg32(xs, src, dst)
    ys = _dec3(acc, xs, dinv, dec_W3, dec_b3.reshape(1, 32), decW4p)
    acc = _agg16(ys, src, dst)
    h2 = _final(acc, ys, dinv, b4p)
    return (h2, mu, sigma)


# speedup vs baseline: 30.0879x; 30.0879x over previous
import functools

import jax
import jax.numpy as jnp
from jax import lax
from jax.experimental import pallas as pl
from jax.experimental.pallas import tpu as pltpu
from jax.experimental.pallas import tpu_sc as plsc

N = 100000
G = 1000
E = 3200000
RB = 2000
NBLK = N // RB

_f32 = jnp.float32


def _row_specs(*dims):
    return [pl.BlockSpec((RB, d), lambda i: (i, 0)) for d in dims]


def _split_spec(d=16):
    return pl.BlockSpec((2, RB, d), lambda i: (0, i, 0))


def _full_spec(shape):
    nd = len(shape)
    return pl.BlockSpec(shape, lambda i: (0,) * nd)


def _dot(a, b):
    return jax.lax.dot_general(a, b, (((1,), (0,)), ((), ())),
                               preferred_element_type=_f32,
                               precision=jax.lax.Precision.DEFAULT)


def _prep_body(dp_ref, x_ref, dinv_ref, xs0_ref):
    deg = dp_ref[0] + dp_ref[1] + 1.0
    dinv = lax.rsqrt(deg)
    dinv_ref[...] = dinv
    xs = x_ref[...] * dinv
    xs0_ref[...] = jnp.concatenate(
        [xs, jnp.zeros((RB, 11), _f32)], axis=1)


def _prep(deg_parts, x):
    return pl.pallas_call(
        _prep_body,
        grid=(NBLK,),
        in_specs=[_split_spec(1)] + _row_specs(5),
        out_specs=_row_specs(1, 16),
        out_shape=[jax.ShapeDtypeStruct((N, 1), _f32),
                   jax.ShapeDtypeStruct((N, 16), _f32)],
    )(deg_parts, x)


def _layer16_body(acc_ref, xs_ref, dinv_ref, W_ref, b_ref, out_ref):
    dinv = dinv_ref[...]
    t = dinv * (acc_ref[0] + acc_ref[1] + xs_ref[...])
    h = jnp.maximum(_dot(t, W_ref[...]) + b_ref[...], 0.0)
    xso = dinv * h
    out_ref[0] = xso[:, :16]
    out_ref[1] = xso[:, 16:]


def _layer16(acc_parts, xs, dinv, Wp, b):
    return pl.pallas_call(
        _layer16_body,
        grid=(NBLK,),
        in_specs=[_split_spec(), *_row_specs(16, 1),
                  _full_spec((16, 32)), _full_spec((1, 32))],
        out_specs=_split_spec(),
        out_shape=jax.ShapeDtypeStruct((2, N, 16), _f32),
    )(acc_parts, xs, dinv, Wp, b)


def _layer32_body(acc_ref, xs_ref, dinv_ref, W_ref, b_ref, out_ref):
    dinv = dinv_ref[...]
    s = jnp.concatenate([acc_ref[0], acc_ref[1]], axis=1)
    xsc = jnp.concatenate([xs_ref[0], xs_ref[1]], axis=1)
    t = dinv * (s + xsc)
    h = jnp.maximum(_dot(t, W_ref[...]) + b_ref[...], 0.0)
    xso = dinv * h
    out_ref[0] = xso[:, :16]
    out_ref[1] = xso[:, 16:]


def _layer32(acc, xs, dinv, W, b):
    return pl.pallas_call(
        _layer32_body,
        grid=(NBLK,),
        in_specs=[_split_spec(), _split_spec(), *_row_specs(1),
                  _full_spec((32, 32)), _full_spec((1, 32))],
        out_specs=_split_spec(),
        out_shape=jax.ShapeDtypeStruct((2, N, 16), _f32),
    )(acc, xs, dinv, W, b)


def _enc4_body(acc_ref, xs_ref, dinv_ref, W_ref, b_ref,
               mW1_ref, mb1_ref, mW2_ref, mb2_ref,
               gW1_ref, gb1_ref, gW2_ref, gb2_ref, out_ref):
    dinv = dinv_ref[...]
    s = jnp.concatenate([acc_ref[0], acc_ref[1]], axis=1)
    xsc = jnp.concatenate([xs_ref[0], xs_ref[1]], axis=1)
    t = dinv * (s + xsc)
    h = jnp.maximum(_dot(t, W_ref[...]) + b_ref[...], 0.0)
    m = _dot(jnp.maximum(_dot(h, mW1_ref[...]) + mb1_ref[...], 0.0),
             mW2_ref[...]) + mb2_ref[...]
    g = _dot(jnp.maximum(_dot(h, gW1_ref[...]) + gb1_ref[...], 0.0),
             gW2_ref[...]) + gb2_ref[...]
    out_ref[...] = jnp.concatenate(
        [m, g, jnp.ones((RB, 1), _f32), jnp.zeros((RB, 9), _f32)], axis=1)


def _enc4_head(acc, xs, dinv, W, b, mW1, mb1, mW2, mb2, gW1, gb1, gW2, gb2):
    return pl.pallas_call(
        _enc4_body,
        grid=(NBLK,),
        in_specs=[_split_spec(), _split_spec(), *_row_specs(1),
                  _full_spec((32, 32)), _full_spec((1, 32)),
                  _full_spec((32, 16)), _full_spec((1, 16)),
                  _full_spec((16, 3)), _full_spec((1, 3)),
                  _full_spec((32, 16)), _full_spec((1, 16)),
                  _full_spec((16, 3)), _full_spec((1, 3))],
        out_specs=_row_specs(16)[0],
        out_shape=jax.ShapeDtypeStruct((N, 16), _f32),
    )(acc, xs, dinv, W, b, mW1, mb1, mW2, mb2, gW1, gb1, gW2, gb2)


def _z_body(pool_ref, eps_ref, mu_ref, sg_ref, zp_ref):
    p = pool_ref[0] + pool_ref[1]
    denom = jnp.maximum(p[:, 6:7], 1.0)
    mu = p[:, 0:3] / denom
    sg = p[:, 3:6] / denom
    z = mu + eps_ref[...] * jnp.exp(0.5 * sg)
    mu_ref[...] = mu
    sg_ref[...] = sg
    zp_ref[...] = jnp.concatenate([z, jnp.zeros((G, 13), _f32)], axis=1)


def _z_kernel(pooled, eps):
    return pl.pallas_call(
        _z_body,
        in_specs=[pl.BlockSpec((2, G, 16), lambda: (0, 0, 0)),
                  pl.BlockSpec((G, 3), lambda: (0, 0))],
        out_specs=[pl.BlockSpec((G, 3), lambda: (0, 0)),
                   pl.BlockSpec((G, 3), lambda: (0, 0)),
                   pl.BlockSpec((G, 16), lambda: (0, 0))],
        out_shape=[jax.ShapeDtypeStruct((G, 3), _f32),
                   jax.ShapeDtypeStruct((G, 3), _f32),
                   jax.ShapeDtypeStruct((G, 16), _f32)],
    )(pooled, eps)


def _dec_head_body(zn_ref, dinv_ref, W1_ref, b1_ref, W2_ref, b2_ref, out_ref):
    h = jnp.maximum(_dot(zn_ref[...], W1_ref[...]) + b1_ref[...], 0.0)
    h2 = jnp.maximum(_dot(h, W2_ref[...]) + b2_ref[...], 0.0)
    xso = dinv_ref[...] * h2
    out_ref[0] = xso[:, :16]
    out_ref[1] = xso[:, 16:]


def _dec_head(zn, dinv, W1p, b1, W2, b2):
    return pl.pallas_call(
        _dec_head_body,
        grid=(NBLK,),
        in_specs=[*_row_specs(16, 1), _full_spec((16, 16)),
                  _full_spec((1, 16)), _full_spec((16, 32)),
                  _full_spec((1, 32))],
        out_specs=_split_spec(),
        out_shape=jax.ShapeDtypeStruct((2, N, 16), _f32),
    )(zn, dinv, W1p, b1, W2, b2)


def _dec3_body(acc_ref, xs_ref, dinv_ref, W_ref, b_ref, W4_ref, out_ref):
    dinv = dinv_ref[...]
    s = jnp.concatenate([acc_ref[0], acc_ref[1]], axis=1)
    xsc = jnp.concatenate([xs_ref[0], xs_ref[1]], axis=1)
    t = dinv * (s + xsc)
    h = jnp.maximum(_dot(t, W_ref[...]) + b_ref[...], 0.0)
    out_ref[...] = _dot(dinv * h, W4_ref[...])


def _dec3(acc, xs, dinv, W3, b3, W4p):
    return pl.pallas_call(
        _dec3_body,
        grid=(NBLK,),
        in_specs=[_split_spec(), _split_spec(), *_row_specs(1),
                  _full_spec((32, 32)), _full_spec((1, 32)),
                  _full_spec((32, 16))],
        out_specs=_row_specs(16)[0],
        out_shape=jax.ShapeDtypeStruct((N, 16), _f32),
    )(acc, xs, dinv, W3, b3, W4p)


def _final_body(acc_ref, ys_ref, dinv_ref, b_ref, out_ref):
    t = dinv_ref[...] * (acc_ref[0] + acc_ref[1] + ys_ref[...]) + b_ref[...]
    out_ref[...] = jnp.maximum(t, 0.0)[:, :5]


def _final(acc_parts, ys, dinv, b4p):
    return pl.pallas_call(
        _final_body,
        grid=(NBLK,),
        in_specs=[_split_spec(), *_row_specs(16, 1), _full_spec((1, 16))],
        out_specs=_row_specs(5)[0],
        out_shape=jax.ShapeDtypeStruct((N, 5), _f32),
    )(acc_parts, ys, dinv, b4p)


E_PAD = 3211264
IROWS = E_PAD // 128
ACC_R = 100352
DUMMY = 100000
CWIN = 8

_mesh = plsc.VectorSubcoreMesh(core_axis_name="c", subcore_axis_name="s")


def _edge_pass(xs, si_pad, di_pad, col_split):
    n_chunks = (IROWS // 16 if col_split else IROWS // 32) // CWIN

    @functools.partial(
        pl.kernel,
        compiler_params=pltpu.CompilerParams(use_tc_tiling_on_sc=False),
        out_type=jax.ShapeDtypeStruct((2, N, 16), _f32),
        mesh=_mesh,
        scratch_types=[
            pltpu.VMEM((CWIN, 128), jnp.int32),
            pltpu.VMEM((CWIN, 128), jnp.int32),
            pltpu.VMEM((CWIN, 128, 16), _f32),
            pltpu.VMEM((128, 16), _f32),
            pltpu.VMEM_SHARED((ACC_R, 16), _f32),
            pltpu.SemaphoreType.DMA,
            pltpu.SemaphoreType.DMA,
            pltpu.SemaphoreType.DMA,
        ],
    )
    def k(xs_hbm, si_hbm, di_hbm, out_hbm,
          si_v, di_v, rows_v, zb_v, acc, sem_i, sem_g, sem_s):
        c = lax.axis_index("c")
        s = lax.axis_index("s")

        @pl.loop(0, 128)
        def _fill(i):
            zb_v[i, :] = jnp.zeros((16,), _f32)

        @pl.loop(0, 49)
        def _zero(i):
            pltpu.sync_copy(zb_v, acc.at[pl.ds(s * 6272 + i * 128, 128)])

        plsc.subcore_barrier()

        base_row = s * 1568 if col_split else (s * 2 + c) * 784

        @pl.loop(0, n_chunks)
        def _chunk(i):
            row0 = base_row + i * CWIN
            ci = pltpu.async_copy(si_hbm.at[pl.ds(row0, CWIN)], si_v, sem_i)
            cd = pltpu.async_copy(di_hbm.at[pl.ds(row0, CWIN)], di_v, sem_i)
            ci.wait()
            cd.wait()
            src_tbl = xs_hbm.at[c] if col_split else xs_hbm
            gs = [pltpu.async_copy(src_tbl.at[si_v.at[j]], rows_v.at[j],
                                   sem_g) for j in range(CWIN)]
            for g in gs:
                g.wait()
            ss = [pltpu.async_copy(rows_v.at[j], acc.at[di_v.at[j]],
                                   sem_s, add=True) for j in range(CWIN)]
            for t in ss:
                t.wait()

        plsc.subcore_barrier()

        @pl.when(s < 15)
        def _drain():
            pltpu.sync_copy(acc.at[pl.ds(s * 6256, 6256)],
                            out_hbm.at[c].at[pl.ds(s * 6256, 6256)])

        @pl.when(s == 15)
        def _drain_tail():
            pltpu.sync_copy(acc.at[pl.ds(93840, 6160)],
                            out_hbm.at[c].at[pl.ds(93840, 6160)])

    return k(xs, si_pad, di_pad)


def _sc_deg(di_pad):
    n_chunks = (IROWS // 32) // CWIN

    @functools.partial(
        pl.kernel,
        compiler_params=pltpu.CompilerParams(use_tc_tiling_on_sc=False),
        out_type=[jax.ShapeDtypeStruct((ACC_R,), _f32),
                  jax.ShapeDtypeStruct((ACC_R,), _f32)],
        mesh=_mesh,
        scratch_types=[
            pltpu.VMEM((CWIN, 128), jnp.int32),
            pltpu.VMEM((128,), _f32),
            pltpu.VMEM((784,), _f32),
            pltpu.VMEM_SHARED((ACC_R,), _f32),
            pltpu.SemaphoreType.DMA,
            pltpu.SemaphoreType.DMA,
        ],
    )
    def k(di_hbm, out0_hbm, out1_hbm, di_v, ones_v, zb_v, acc, sem_i, sem_s):
        c = lax.axis_index("c")
        s = lax.axis_index("s")

        @pl.loop(0, 8)
        def _fill1(i):
            ones_v[pl.ds(i * 16, 16)] = jnp.ones((16,), _f32)

        @pl.loop(0, 49)
        def _fill0(i):
            zb_v[pl.ds(i * 16, 16)] = jnp.zeros((16,), _f32)

        @pl.loop(0, 8)
        def _zero(i):
            pltpu.sync_copy(zb_v, acc.at[pl.ds(s * 6272 + i * 784, 784)])

        plsc.subcore_barrier()
        base_row = (s * 2 + c) * 784

        @pl.loop(0, n_chunks)
        def _chunk(i):
            row0 = base_row + i * CWIN
            pltpu.async_copy(di_hbm.at[pl.ds(row0, CWIN)], di_v, sem_i).wait()
            ss = [pltpu.async_copy(ones_v, acc.at[di_v.at[j]], sem_s,
                                   add=True) for j in range(CWIN)]
            for t in ss:
                t.wait()

        plsc.subcore_barrier()

        @pl.when(c == 0)
        def _drain0():
            pltpu.sync_copy(acc.at[pl.ds(s * 6272, 6272)],
                            out0_hbm.at[pl.ds(s * 6272, 6272)])

        @pl.when(c == 1)
        def _drain1():
            pltpu.sync_copy(acc.at[pl.ds(s * 6272, 6272)],
                            out1_hbm.at[pl.ds(s * 6272, 6272)])

    return k(di_pad)


NP = 131072
GDUM = 1000


def _sc_pool(combined_pad, bi_rows):
    @functools.partial(
        pl.kernel,
        compiler_params=pltpu.CompilerParams(use_tc_tiling_on_sc=False),
        out_type=jax.ShapeDtypeStruct((2, G, 16), _f32),
        mesh=_mesh,
        scratch_types=[
            pltpu.VMEM((32, 128), jnp.int32),
            pltpu.VMEM((4096, 16), _f32),
            pltpu.VMEM((64, 16), _f32),
            pltpu.VMEM_SHARED((1024, 16), _f32),
            pltpu.SemaphoreType.DMA,
            pltpu.SemaphoreType.DMA,
        ],
    )
    def k(comb_hbm, bi_hbm, out_hbm, bi_v, rows_v, zb_v, acc, sem_i, sem_s):
        c = lax.axis_index("c")
        s = lax.axis_index("s")

        @pl.loop(0, 64)
        def _fill(i):
            zb_v[i, :] = jnp.zeros((16,), _f32)

        pltpu.sync_copy(zb_v, acc.at[pl.ds(s * 64, 64)])
        plsc.subcore_barrier()

        w = s * 2 + c
        ci = pltpu.async_copy(bi_hbm.at[pl.ds(w * 32, 32)], bi_v, sem_i)
        cr = pltpu.async_copy(comb_hbm.at[pl.ds(w * 4096, 4096)], rows_v,
                              sem_i)
        ci.wait()
        cr.wait()
        ss = [pltpu.async_copy(rows_v.at[pl.ds(j * 128, 128)],
                               acc.at[bi_v.at[j]], sem_s, add=True)
              for j in range(32)]
        for t in ss:
            t.wait()
        plsc.subcore_barrier()

        @pl.when(s == 0)
        def _drain():
            pltpu.sync_copy(acc.at[pl.ds(0, G)], out_hbm.at[c])

    return k(combined_pad, bi_rows)


def _sc_zn(zpad, bi_rows):
    @functools.partial(
        pl.kernel,
        compiler_params=pltpu.CompilerParams(use_tc_tiling_on_sc=False),
        out_type=jax.ShapeDtypeStruct((NP, 16), _f32),
        mesh=_mesh,
        scratch_types=[
            pltpu.VMEM((32, 128), jnp.int32),
            pltpu.VMEM((4096, 16), _f32),
            pltpu.SemaphoreType.DMA,
            pltpu.SemaphoreType.DMA,
        ],
    )
    def k(z_hbm, bi_hbm, out_hbm, bi_v, rows_v, sem_i, sem_g):
        c = lax.axis_index("c")
        s = lax.axis_index("s")
        w = s * 2 + c
        pltpu.async_copy(bi_hbm.at[pl.ds(w * 32, 32)], bi_v, sem_i).wait()
        gs = [pltpu.async_copy(z_hbm.at[bi_v.at[j]],
                               rows_v.at[pl.ds(j * 128, 128)], sem_g)
              for j in range(32)]
        for g in gs:
            g.wait()
        pltpu.sync_copy(rows_v, out_hbm.at[pl.ds(w * 4096, 4096)])

    return k(zpad, bi_rows)


def kernel(x, edge_index, batch_index,
           enc_W1, enc_b1, enc_W2, enc_b2, enc_W3, enc_b3, enc_W4, enc_b4,
           mu_W1, mu_b1, mu_W2, mu_b2,
           sg_W1, sg_b1, sg_W2, sg_b2,
           un_W1, un_b1, un_W2, un_b2,
           dec_W1, dec_b1, dec_W2, dec_b2, dec_W3, dec_b3, dec_W4, dec_b4):
    src = edge_index[0].astype(jnp.int32)
    dst = edge_index[1].astype(jnp.int32)

    W1p = jnp.zeros((16, 32), _f32).at[:5].set(enc_W1)
    unW1p = jnp.zeros((16, 16), _f32).at[:3].set(un_W1)
    decW4p = jnp.zeros((32, 16), _f32).at[:, :5].set(dec_W4)
    b4p = jnp.zeros((1, 16), _f32).at[0, :5].set(dec_b4)
    eps = jax.random.normal(jax.random.key(42), (G, 3), dtype=_f32)

    npad = E_PAD - E
    si_pad = jnp.concatenate(
        [src, jnp.zeros((npad,), jnp.int32)]).reshape(IROWS, 128)
    di_pad = jnp.concatenate(
        [dst, jnp.full((npad,), DUMMY, jnp.int32)]).reshape(IROWS, 128)
    bi_pad = jnp.concatenate(
        [batch_index.astype(jnp.int32),
         jnp.full((NP - N,), GDUM, jnp.int32)]).reshape(1024, 128)

    deg0, deg1 = _sc_deg(di_pad)
    deg_parts = jnp.stack([deg0[:N], deg1[:N]]).reshape(2, N, 1)
    dinv, xs0 = _prep(deg_parts, x)

    acc = _edge_pass(xs0, si_pad, di_pad, col_split=False)
    xs = _layer16(acc, xs0, dinv, W1p, enc_b1.reshape(1, 32))
    for W, b in ((enc_W2, enc_b2), (enc_W3, enc_b3)):
        acc = _edge_pass(xs, si_pad, di_pad, col_split=True)
        xs = _layer32(acc, xs, dinv, W, b.reshape(1, 32))
    acc = _edge_pass(xs, si_pad, di_pad, col_split=True)
    combined = _enc4_head(acc, xs, dinv, enc_W4, enc_b4.reshape(1, 32),
                          mu_W1, mu_b1.reshape(1, 16), mu_W2,
                          mu_b2.reshape(1, 3),
                          sg_W1, sg_b1.reshape(1, 16), sg_W2,
                          sg_b2.reshape(1, 3))

    combined_pad = jnp.concatenate(
        [combined, jnp.zeros((NP - N, 16), _f32)])
    pooled = _sc_pool(combined_pad, bi_pad)
    mu, sigma, zpad = _z_kernel(pooled, eps)

    zpad2 = jnp.concatenate([zpad, jnp.zeros((8, 16), _f32)])
    zn = _sc_zn(zpad2, bi_pad)[:N]
    xs = _dec_head(zn, dinv, unW1p, un_b1.reshape(1, 16), un_W2,
                   un_b2.reshape(1, 32))
    for W, b in ((dec_W1, dec_b1), (dec_W2, dec_b2)):
        acc = _edge_pass(xs, si_pad, di_pad, col_split=True)
        xs = _layer32(acc, xs, dinv, W, b.reshape(1, 32))
    acc = _edge_pass(xs, si_pad, di_pad, col_split=True)
    ys = _dec3(acc, xs, dinv, dec_W3, dec_b3.reshape(1, 32), decW4p)
    acc = _edge_pass(ys, si_pad, di_pad, col_split=False)
    h2 = _final(acc, ys, dinv, b4p)
    return (h2, mu, sigma)


# speedup vs baseline: 33.4054x; 1.1103x over previous
import functools

import jax
import jax.numpy as jnp
from jax import lax
from jax.experimental import pallas as pl
from jax.experimental.pallas import tpu as pltpu
from jax.experimental.pallas import tpu_sc as plsc

N = 100000
G = 1000
E = 3200000
RB = 2000
NBLK = N // RB

_f32 = jnp.float32


def _row_specs(*dims):
    return [pl.BlockSpec((RB, d), lambda i: (i, 0)) for d in dims]


def _split_spec(d=16):
    return pl.BlockSpec((2, RB, d), lambda i: (0, i, 0))


def _full_spec(shape):
    nd = len(shape)
    return pl.BlockSpec(shape, lambda i: (0,) * nd)


def _dot(a, b):
    return jax.lax.dot_general(a, b, (((1,), (0,)), ((), ())),
                               preferred_element_type=_f32,
                               precision=jax.lax.Precision.DEFAULT)


def _prep_body(dp_ref, x_ref, dinv_ref, xs0_ref):
    deg = dp_ref[0] + dp_ref[1] + 1.0
    dinv = lax.rsqrt(deg)
    dinv_ref[...] = dinv
    xs = x_ref[...] * dinv
    xs0_ref[...] = jnp.concatenate(
        [xs, jnp.zeros((RB, 11), _f32)], axis=1)


def _prep(deg_parts, x):
    return pl.pallas_call(
        _prep_body,
        grid=(NBLK,),
        in_specs=[_split_spec(1)] + _row_specs(5),
        out_specs=_row_specs(1, 16),
        out_shape=[jax.ShapeDtypeStruct((N, 1), _f32),
                   jax.ShapeDtypeStruct((N, 16), _f32)],
    )(deg_parts, x)


def _layer16_body(acc_ref, xs_ref, dinv_ref, W_ref, b_ref, out_ref):
    dinv = dinv_ref[...]
    t = dinv * (acc_ref[0] + acc_ref[1] + xs_ref[...])
    h = jnp.maximum(_dot(t, W_ref[...]) + b_ref[...], 0.0)
    xso = dinv * h
    out_ref[0] = xso[:, :16]
    out_ref[1] = xso[:, 16:]


def _layer16(acc_parts, xs, dinv, Wp, b):
    return pl.pallas_call(
        _layer16_body,
        grid=(NBLK,),
        in_specs=[_split_spec(), *_row_specs(16, 1),
                  _full_spec((16, 32)), _full_spec((1, 32))],
        out_specs=_split_spec(),
        out_shape=jax.ShapeDtypeStruct((2, N, 16), _f32),
    )(acc_parts, xs, dinv, Wp, b)


def _layer32_body(acc_ref, xs_ref, dinv_ref, W_ref, b_ref, out_ref):
    dinv = dinv_ref[...]
    s = jnp.concatenate([acc_ref[0], acc_ref[1]], axis=1)
    xsc = jnp.concatenate([xs_ref[0], xs_ref[1]], axis=1)
    t = dinv * (s + xsc)
    h = jnp.maximum(_dot(t, W_ref[...]) + b_ref[...], 0.0)
    xso = dinv * h
    out_ref[0] = xso[:, :16]
    out_ref[1] = xso[:, 16:]


def _layer32(acc, xs, dinv, W, b):
    return pl.pallas_call(
        _layer32_body,
        grid=(NBLK,),
        in_specs=[_split_spec(), _split_spec(), *_row_specs(1),
                  _full_spec((32, 32)), _full_spec((1, 32))],
        out_specs=_split_spec(),
        out_shape=jax.ShapeDtypeStruct((2, N, 16), _f32),
    )(acc, xs, dinv, W, b)


def _enc4_body(acc_ref, xs_ref, dinv_ref, W_ref, b_ref,
               mW1_ref, mb1_ref, mW2_ref, mb2_ref,
               gW1_ref, gb1_ref, gW2_ref, gb2_ref, out_ref):
    dinv = dinv_ref[...]
    s = jnp.concatenate([acc_ref[0], acc_ref[1]], axis=1)
    xsc = jnp.concatenate([xs_ref[0], xs_ref[1]], axis=1)
    t = dinv * (s + xsc)
    h = jnp.maximum(_dot(t, W_ref[...]) + b_ref[...], 0.0)
    m = _dot(jnp.maximum(_dot(h, mW1_ref[...]) + mb1_ref[...], 0.0),
             mW2_ref[...]) + mb2_ref[...]
    g = _dot(jnp.maximum(_dot(h, gW1_ref[...]) + gb1_ref[...], 0.0),
             gW2_ref[...]) + gb2_ref[...]
    out_ref[...] = jnp.concatenate(
        [m, g, jnp.ones((RB, 1), _f32), jnp.zeros((RB, 9), _f32)], axis=1)


def _enc4_head(acc, xs, dinv, W, b, mW1, mb1, mW2, mb2, gW1, gb1, gW2, gb2):
    return pl.pallas_call(
        _enc4_body,
        grid=(NBLK,),
        in_specs=[_split_spec(), _split_spec(), *_row_specs(1),
                  _full_spec((32, 32)), _full_spec((1, 32)),
                  _full_spec((32, 16)), _full_spec((1, 16)),
                  _full_spec((16, 3)), _full_spec((1, 3)),
                  _full_spec((32, 16)), _full_spec((1, 16)),
                  _full_spec((16, 3)), _full_spec((1, 3))],
        out_specs=_row_specs(16)[0],
        out_shape=jax.ShapeDtypeStruct((N, 16), _f32),
    )(acc, xs, dinv, W, b, mW1, mb1, mW2, mb2, gW1, gb1, gW2, gb2)


def _z_body(pool_ref, eps_ref, mu_ref, sg_ref, zp_ref):
    p = pool_ref[0] + pool_ref[1]
    denom = jnp.maximum(p[:, 6:7], 1.0)
    mu = p[:, 0:3] / denom
    sg = p[:, 3:6] / denom
    z = mu + eps_ref[...] * jnp.exp(0.5 * sg)
    mu_ref[...] = mu
    sg_ref[...] = sg
    zp_ref[...] = jnp.concatenate([z, jnp.zeros((G, 13), _f32)], axis=1)


def _z_kernel(pooled, eps):
    return pl.pallas_call(
        _z_body,
        in_specs=[pl.BlockSpec((2, G, 16), lambda: (0, 0, 0)),
                  pl.BlockSpec((G, 3), lambda: (0, 0))],
        out_specs=[pl.BlockSpec((G, 3), lambda: (0, 0)),
                   pl.BlockSpec((G, 3), lambda: (0, 0)),
                   pl.BlockSpec((G, 16), lambda: (0, 0))],
        out_shape=[jax.ShapeDtypeStruct((G, 3), _f32),
                   jax.ShapeDtypeStruct((G, 3), _f32),
                   jax.ShapeDtypeStruct((G, 16), _f32)],
    )(pooled, eps)


def _dec_head_body(zn_ref, dinv_ref, W1_ref, b1_ref, W2_ref, b2_ref, out_ref):
    h = jnp.maximum(_dot(zn_ref[...], W1_ref[...]) + b1_ref[...], 0.0)
    h2 = jnp.maximum(_dot(h, W2_ref[...]) + b2_ref[...], 0.0)
    xso = dinv_ref[...] * h2
    out_ref[0] = xso[:, :16]
    out_ref[1] = xso[:, 16:]


def _dec_head(zn, dinv, W1p, b1, W2, b2):
    return pl.pallas_call(
        _dec_head_body,
        grid=(NBLK,),
        in_specs=[*_row_specs(16, 1), _full_spec((16, 16)),
                  _full_spec((1, 16)), _full_spec((16, 32)),
                  _full_spec((1, 32))],
        out_specs=_split_spec(),
        out_shape=jax.ShapeDtypeStruct((2, N, 16), _f32),
    )(zn, dinv, W1p, b1, W2, b2)


def _dec3_body(acc_ref, xs_ref, dinv_ref, W_ref, b_ref, W4_ref, out_ref):
    dinv = dinv_ref[...]
    s = jnp.concatenate([acc_ref[0], acc_ref[1]], axis=1)
    xsc = jnp.concatenate([xs_ref[0], xs_ref[1]], axis=1)
    t = dinv * (s + xsc)
    h = jnp.maximum(_dot(t, W_ref[...]) + b_ref[...], 0.0)
    out_ref[...] = _dot(dinv * h, W4_ref[...])


def _dec3(acc, xs, dinv, W3, b3, W4p):
    return pl.pallas_call(
        _dec3_body,
        grid=(NBLK,),
        in_specs=[_split_spec(), _split_spec(), *_row_specs(1),
                  _full_spec((32, 32)), _full_spec((1, 32)),
                  _full_spec((32, 16))],
        out_specs=_row_specs(16)[0],
        out_shape=jax.ShapeDtypeStruct((N, 16), _f32),
    )(acc, xs, dinv, W3, b3, W4p)


def _final_body(acc_ref, ys_ref, dinv_ref, b_ref, out_ref):
    t = dinv_ref[...] * (acc_ref[0] + acc_ref[1] + ys_ref[...]) + b_ref[...]
    out_ref[...] = jnp.maximum(t, 0.0)[:, :5]


def _final(acc_parts, ys, dinv, b4p):
    return pl.pallas_call(
        _final_body,
        grid=(NBLK,),
        in_specs=[_split_spec(), *_row_specs(16, 1), _full_spec((1, 16))],
        out_specs=_row_specs(5)[0],
        out_shape=jax.ShapeDtypeStruct((N, 5), _f32),
    )(acc_parts, ys, dinv, b4p)


E_PAD = 3211264
IROWS = E_PAD // 128
ACC_R = 100352
DUMMY = 100000
CWIN = 4

_mesh = plsc.VectorSubcoreMesh(core_axis_name="c", subcore_axis_name="s")


def _edge_pass(xs, si_pad, di_pad, col_split):
    n_chunks = (IROWS // 16 if col_split else IROWS // 32) // CWIN

    @functools.partial(
        pl.kernel,
        compiler_params=pltpu.CompilerParams(use_tc_tiling_on_sc=False),
        out_type=jax.ShapeDtypeStruct((2, N, 16), _f32),
        mesh=_mesh,
        scratch_types=[
            pltpu.VMEM((2, CWIN, 128), jnp.int32),
            pltpu.VMEM((2, CWIN, 128), jnp.int32),
            pltpu.VMEM((2, CWIN, 128, 16), _f32),
            pltpu.VMEM((128, 16), _f32),
            pltpu.VMEM_SHARED((ACC_R, 16), _f32),
            pltpu.SemaphoreType.DMA,
            pltpu.SemaphoreType.DMA,
            pltpu.SemaphoreType.DMA,
            pltpu.SemaphoreType.DMA,
            pltpu.SemaphoreType.DMA,
        ],
    )
    def k(xs_hbm, si_hbm, di_hbm, out_hbm,
          si_v, di_v, rows_v, zb_v, acc,
          sem_i0, sem_i1, sem_g, sem_s0, sem_s1):
        c = lax.axis_index("c")
        s = lax.axis_index("s")

        @pl.loop(0, 128)
        def _fill(i):
            zb_v[i, :] = jnp.zeros((16,), _f32)

        @pl.loop(0, 49)
        def _zero(i):
            pltpu.sync_copy(zb_v, acc.at[pl.ds(s * 6272 + i * 128, 128)])

        plsc.subcore_barrier()

        base_row = s * 1568 if col_split else (s * 2 + c) * 784
        sem_i = (sem_i0, sem_i1)
        sem_s = (sem_s0, sem_s1)
        src_tbl = xs_hbm.at[c] if col_split else xs_hbm
        last = n_chunks - 1

        def stage_idx(i, b):
            row0 = base_row + jnp.minimum(i, last) * CWIN
            pltpu.async_copy(si_hbm.at[pl.ds(row0, CWIN)], si_v.at[b],
                             sem_i[b])
            pltpu.async_copy(di_hbm.at[pl.ds(row0, CWIN)], di_v.at[b],
                             sem_i[b])

        def wait_idx(i, b):
            row0 = base_row + jnp.minimum(i, last) * CWIN
            pltpu.make_async_copy(si_hbm.at[pl.ds(row0, CWIN)], si_v.at[b],
                                  sem_i[b]).wait()
            pltpu.make_async_copy(di_hbm.at[pl.ds(row0, CWIN)], di_v.at[b],
                                  sem_i[b]).wait()

        def wait_scat(b):
            for j in range(CWIN):
                pltpu.make_async_copy(rows_v.at[b].at[j],
                                      acc.at[di_v.at[b].at[j]],
                                      sem_s[b]).wait()

        def do_chunk(i, b, first):
            if not first:
                wait_scat(b)
            wait_idx(i, b)
            gs = [pltpu.async_copy(src_tbl.at[si_v.at[b].at[j]],
                                   rows_v.at[b].at[j], sem_g)
                  for j in range(CWIN)]
            for g in gs:
                g.wait()
            for j in range(CWIN):
                pltpu.async_copy(rows_v.at[b].at[j],
                                 acc.at[di_v.at[b].at[j]], sem_s[b],
                                 add=True)
            stage_idx(i + 2, b)

        stage_idx(0, 0)
        stage_idx(1, 1)
        do_chunk(0, 0, True)
        do_chunk(1, 1, True)

        @pl.loop(1, n_chunks // 2)
        def _pair(p):
            do_chunk(2 * p, 0, False)
            do_chunk(2 * p + 1, 1, False)

        wait_scat(0)
        wait_scat(1)
        wait_idx(n_chunks, 0)
        wait_idx(n_chunks + 1, 1)
        plsc.subcore_barrier()

        @pl.when(s < 15)
        def _drain():
            pltpu.sync_copy(acc.at[pl.ds(s * 6256, 6256)],
                            out_hbm.at[c].at[pl.ds(s * 6256, 6256)])

        @pl.when(s == 15)
        def _drain_tail():
            pltpu.sync_copy(acc.at[pl.ds(93840, 6160)],
                            out_hbm.at[c].at[pl.ds(93840, 6160)])

    return k(xs, si_pad, di_pad)


def _sc_deg(di_pad):
    n_chunks = (IROWS // 32) // CWIN

    @functools.partial(
        pl.kernel,
        compiler_params=pltpu.CompilerParams(use_tc_tiling_on_sc=False),
        out_type=[jax.ShapeDtypeStruct((ACC_R,), _f32),
                  jax.ShapeDtypeStruct((ACC_R,), _f32)],
        mesh=_mesh,
        scratch_types=[
            pltpu.VMEM((CWIN, 128), jnp.int32),
            pltpu.VMEM((128,), _f32),
            pltpu.VMEM((784,), _f32),
            pltpu.VMEM_SHARED((ACC_R,), _f32),
            pltpu.SemaphoreType.DMA,
            pltpu.SemaphoreType.DMA,
        ],
    )
    def k(di_hbm, out0_hbm, out1_hbm, di_v, ones_v, zb_v, acc, sem_i, sem_s):
        c = lax.axis_index("c")
        s = lax.axis_index("s")

        @pl.loop(0, 8)
        def _fill1(i):
            ones_v[pl.ds(i * 16, 16)] = jnp.ones((16,), _f32)

        @pl.loop(0, 49)
        def _fill0(i):
            zb_v[pl.ds(i * 16, 16)] = jnp.zeros((16,), _f32)

        @pl.loop(0, 8)
        def _zero(i):
            pltpu.sync_copy(zb_v, acc.at[pl.ds(s * 6272 + i * 784, 784)])

        plsc.subcore_barrier()
        base_row = (s * 2 + c) * 784

        @pl.loop(0, n_chunks)
        def _chunk(i):
            row0 = base_row + i * CWIN
            pltpu.async_copy(di_hbm.at[pl.ds(row0, CWIN)], di_v, sem_i).wait()
            ss = [pltpu.async_copy(ones_v, acc.at[di_v.at[j]], sem_s,
                                   add=True) for j in range(CWIN)]
            for t in ss:
                t.wait()

        plsc.subcore_barrier()

        @pl.when(c == 0)
        def _drain0():
            pltpu.sync_copy(acc.at[pl.ds(s * 6272, 6272)],
                            out0_hbm.at[pl.ds(s * 6272, 6272)])

        @pl.when(c == 1)
        def _drain1():
            pltpu.sync_copy(acc.at[pl.ds(s * 6272, 6272)],
                            out1_hbm.at[pl.ds(s * 6272, 6272)])

    return k(di_pad)


NP = 131072
GDUM = 1000


def _sc_pool(combined_pad, bi_rows):
    @functools.partial(
        pl.kernel,
        compiler_params=pltpu.CompilerParams(use_tc_tiling_on_sc=False),
        out_type=jax.ShapeDtypeStruct((2, G, 16), _f32),
        mesh=_mesh,
        scratch_types=[
            pltpu.VMEM((32, 128), jnp.int32),
            pltpu.VMEM((4096, 16), _f32),
            pltpu.VMEM((64, 16), _f32),
            pltpu.VMEM_SHARED((1024, 16), _f32),
            pltpu.SemaphoreType.DMA,
            pltpu.SemaphoreType.DMA,
        ],
    )
    def k(comb_hbm, bi_hbm, out_hbm, bi_v, rows_v, zb_v, acc, sem_i, sem_s):
        c = lax.axis_index("c")
        s = lax.axis_index("s")

        @pl.loop(0, 64)
        def _fill(i):
            zb_v[i, :] = jnp.zeros((16,), _f32)

        pltpu.sync_copy(zb_v, acc.at[pl.ds(s * 64, 64)])
        plsc.subcore_barrier()

        w = s * 2 + c
        ci = pltpu.async_copy(bi_hbm.at[pl.ds(w * 32, 32)], bi_v, sem_i)
        cr = pltpu.async_copy(comb_hbm.at[pl.ds(w * 4096, 4096)], rows_v,
                              sem_i)
        ci.wait()
        cr.wait()
        ss = [pltpu.async_copy(rows_v.at[pl.ds(j * 128, 128)],
                               acc.at[bi_v.at[j]], sem_s, add=True)
              for j in range(32)]
        for t in ss:
            t.wait()
        plsc.subcore_barrier()

        @pl.when(s == 0)
        def _drain():
            pltpu.sync_copy(acc.at[pl.ds(0, G)], out_hbm.at[c])

    return k(combined_pad, bi_rows)


def _sc_zn(zpad, bi_rows):
    @functools.partial(
        pl.kernel,
        compiler_params=pltpu.CompilerParams(use_tc_tiling_on_sc=False),
        out_type=jax.ShapeDtypeStruct((NP, 16), _f32),
        mesh=_mesh,
        scratch_types=[
            pltpu.VMEM((32, 128), jnp.int32),
            pltpu.VMEM((4096, 16), _f32),
            pltpu.SemaphoreType.DMA,
            pltpu.SemaphoreType.DMA,
        ],
    )
    def k(z_hbm, bi_hbm, out_hbm, bi_v, rows_v, sem_i, sem_g):
        c = lax.axis_index("c")
        s = lax.axis_index("s")
        w = s * 2 + c
        pltpu.async_copy(bi_hbm.at[pl.ds(w * 32, 32)], bi_v, sem_i).wait()
        gs = [pltpu.async_copy(z_hbm.at[bi_v.at[j]],
                               rows_v.at[pl.ds(j * 128, 128)], sem_g)
              for j in range(32)]
        for g in gs:
            g.wait()
        pltpu.sync_copy(rows_v, out_hbm.at[pl.ds(w * 4096, 4096)])

    return k(zpad, bi_rows)


def kernel(x, edge_index, batch_index,
           enc_W1, enc_b1, enc_W2, enc_b2, enc_W3, enc_b3, enc_W4, enc_b4,
           mu_W1, mu_b1, mu_W2, mu_b2,
           sg_W1, sg_b1, sg_W2, sg_b2,
           un_W1, un_b1, un_W2, un_b2,
           dec_W1, dec_b1, dec_W2, dec_b2, dec_W3, dec_b3, dec_W4, dec_b4):
    src = edge_index[0].astype(jnp.int32)
    dst = edge_index[1].astype(jnp.int32)

    W1p = jnp.zeros((16, 32), _f32).at[:5].set(enc_W1)
    unW1p = jnp.zeros((16, 16), _f32).at[:3].set(un_W1)
    decW4p = jnp.zeros((32, 16), _f32).at[:, :5].set(dec_W4)
    b4p = jnp.zeros((1, 16), _f32).at[0, :5].set(dec_b4)
    eps = jax.random.normal(jax.random.key(42), (G, 3), dtype=_f32)

    npad = E_PAD - E
    si_pad = jnp.concatenate(
        [src, jnp.zeros((npad,), jnp.int32)]).reshape(IROWS, 128)
    di_pad = jnp.concatenate(
        [dst, jnp.full((npad,), DUMMY, jnp.int32)]).reshape(IROWS, 128)
    bi_pad = jnp.concatenate(
        [batch_index.astype(jnp.int32),
         jnp.full((NP - N,), GDUM, jnp.int32)]).reshape(1024, 128)

    deg0, deg1 = _sc_deg(di_pad)
    deg_parts = jnp.stack([deg0[:N], deg1[:N]]).reshape(2, N, 1)
    dinv, xs0 = _prep(deg_parts, x)

    acc = _edge_pass(xs0, si_pad, di_pad, col_split=False)
    xs = _layer16(acc, xs0, dinv, W1p, enc_b1.reshape(1, 32))
    for W, b in ((enc_W2, enc_b2), (enc_W3, enc_b3)):
        acc = _edge_pass(xs, si_pad, di_pad, col_split=True)
        xs = _layer32(acc, xs, dinv, W, b.reshape(1, 32))
    acc = _edge_pass(xs, si_pad, di_pad, col_split=True)
    combined = _enc4_head(acc, xs, dinv, enc_W4, enc_b4.reshape(1, 32),
                          mu_W1, mu_b1.reshape(1, 16), mu_W2,
                          mu_b2.reshape(1, 3),
                          sg_W1, sg_b1.reshape(1, 16), sg_W2,
                          sg_b2.reshape(1, 3))

    combined_pad = jnp.concatenate(
        [combined, jnp.zeros((NP - N, 16), _f32)])
    pooled = _sc_pool(combined_pad, bi_pad)
    mu, sigma, zpad = _z_kernel(pooled, eps)

    zpad2 = jnp.concatenate([zpad, jnp.zeros((8, 16), _f32)])
    zn = _sc_zn(zpad2, bi_pad)[:N]
    xs = _dec_head(zn, dinv, unW1p, un_b1.reshape(1, 16), un_W2,
                   un_b2.reshape(1, 32))
    for W, b in ((dec_W1, dec_b1), (dec_W2, dec_b2)):
        acc = _edge_pass(xs, si_pad, di_pad, col_split=True)
        xs = _layer32(acc, xs, dinv, W, b.reshape(1, 32))
    acc = _edge_pass(xs, si_pad, di_pad, col_split=True)
    ys = _dec3(acc, xs, dinv, dec_W3, dec_b3.reshape(1, 32), decW4p)
    acc = _edge_pass(ys, si_pad, di_pad, col_split=False)
    h2 = _final(acc, ys, dinv, b4p)
    return (h2, mu, sigma)
